# Initial kernel scaffold; baseline (speedup 1.0000x reference)
import jax
import jax.numpy as jnp
from jax.experimental import pallas as pl


def kernel(x, u, batch, W1, b1, g1, be1, W2, b2, g2, be2):
    raise NotImplementedError("write your pallas kernel here")


# speedup vs baseline: 12.8125x; 12.8125x over previous
import functools

import jax
import jax.numpy as jnp
from jax import lax
from jax.experimental import pallas as pl
from jax.experimental.pallas import tpu as pltpu
from jax.experimental.pallas import tpu_sc as plsc

N, D, B, U, HS = 320000, 128, 16, 64, 256
EPS = 1e-5
L = 16
KD = D // L
NC, NS = 2, 16
NW = NC * NS
ROWS_PER_TILE = N // NW
CHUNK_ROWS = 400
NCHUNK = ROWS_PER_TILE // CHUNK_ROWS
GROUPS = CHUNK_ROWS // L

_NEG_INF = float("-inf")


def _pool_body(x_hbm, ids_hbm, omax_hbm, osum_hbm, ocnt_hbm,
               ids_v, xbuf, accm, accs, accc):
    wid = lax.axis_index("c") * NS + lax.axis_index("s")
    base = wid * ROWS_PER_TILE

    neg = jnp.full((L,), _NEG_INF, jnp.float32)
    zero = jnp.zeros((L,), jnp.float32)
    for b in range(B):
        for k in range(KD):
            accm[pl.ds((b * KD + k) * L, L)] = neg
            accs[pl.ds((b * KD + k) * L, L)] = zero
    accc[...] = zero

    pltpu.sync_copy(ids_hbm.at[pl.ds(base, ROWS_PER_TILE)], ids_v)

    iota = lax.iota(jnp.int32, L)

    def _merge_rows(seg, m_regs, s_regs, nrows):
        off = seg * D
        for k in range(KD):
            sl = pl.ds(off + k * L, L)
            accm[sl] = jnp.maximum(accm[sl], m_regs[k])
            accs[sl] = accs[sl] + s_regs[k]
        accc[...] = accc[...] + jnp.where(iota == seg, nrows, 0.0)

    def _group(c, g):
        row0 = c * CHUNK_ROWS + g * L
        ids = ids_v[pl.ds(row0, L)]
        s_first = ids[0]
        s_last = ids[L - 1]

        def uniform(_):
            lrow = g * L * D
            m_regs = [xbuf[pl.ds(lrow + k * L, L)] for k in range(KD)]
            s_regs = list(m_regs)
            for r in range(1, L):
                roff = lrow + r * D
                for k in range(KD):
                    v = xbuf[pl.ds(roff + k * L, L)]
                    m_regs[k] = jnp.maximum(m_regs[k], v)
                    s_regs[k] = s_regs[k] + v
            _merge_rows(s_first, m_regs, s_regs, jnp.float32(L))
            return 0

        def mixed(_):
            for r in range(L):
                seg = ids[r]
                roff = (g * L + r) * D
                regs = [xbuf[pl.ds(roff + k * L, L)] for k in range(KD)]
                _merge_rows(seg, regs, regs, jnp.float32(1))
            return 0

        lax.cond(s_first == s_last, uniform, mixed, 0)
        return g

    def chunk_body(c, carry):
        pltpu.sync_copy(
            x_hbm.at[pl.ds((base + c * CHUNK_ROWS) * D, CHUNK_ROWS * D)],
            xbuf)
        def g_body(g, _c):
            _group(c, g)
            return _c
        lax.fori_loop(0, GROUPS, g_body, 0)
        return carry

    lax.fori_loop(0, NCHUNK, chunk_body, 0)

    pltpu.sync_copy(accm, omax_hbm.at[pl.ds(wid * B * D, B * D)])
    pltpu.sync_copy(accs, osum_hbm.at[pl.ds(wid * B * D, B * D)])
    pltpu.sync_copy(accc, ocnt_hbm.at[pl.ds(wid * B, B)])


_pool = pl.kernel(
    _pool_body,
    out_type=(
        jax.ShapeDtypeStruct((NW * B * D,), jnp.float32),
        jax.ShapeDtypeStruct((NW * B * D,), jnp.float32),
        jax.ShapeDtypeStruct((NW * B,), jnp.float32),
    ),
    mesh=plsc.VectorSubcoreMesh(
        core_axis_name="c", subcore_axis_name="s",
        num_cores=NC, num_subcores=NS),
    scratch_types=[
        pltpu.VMEM((ROWS_PER_TILE,), jnp.int32),
        pltpu.VMEM((CHUNK_ROWS * D,), jnp.float32),
        pltpu.VMEM((B * D,), jnp.float32),
        pltpu.VMEM((B * D,), jnp.float32),
        pltpu.VMEM((B,), jnp.float32),
    ],
)


def _mlp_body(u_ref, pmax_ref, psum_ref, pcnt_ref,
              W1t_ref, b1_ref, g1_ref, be1_ref,
              W2t_ref, b2_ref, g2_ref, be2_ref, o_ref):
    smax = jnp.max(pmax_ref[...], axis=0)
    ssum = jnp.sum(psum_ref[...], axis=0)
    cnt = jnp.sum(pcnt_ref[...], axis=0)
    smean = ssum / jnp.maximum(cnt, 1.0)[:, None]
    out = jnp.concatenate([u_ref[...], smax, smean, ssum], axis=1)
    h = lax.dot_general(out, W1t_ref[...], (((1,), (0,)), ((), ())),
                        preferred_element_type=jnp.float32) + b1_ref[...]
    h = (h / jnp.sqrt(1.0 + EPS)) * g1_ref[...] + be1_ref[...]
    h = jnp.maximum(h, 0.0)
    h = lax.dot_general(h, W2t_ref[...], (((1,), (0,)), ((), ())),
                        preferred_element_type=jnp.float32) + b2_ref[...]
    o_ref[...] = (h / jnp.sqrt(1.0 + EPS)) * g2_ref[...] + be2_ref[...]


_mlp = pl.pallas_call(
    _mlp_body,
    out_shape=jax.ShapeDtypeStruct((B, HS), jnp.float32),
)


@jax.jit
def kernel(x, u, batch, W1, b1, g1, be1, W2, b2, g2, be2):
    pmax, psum, pcnt = _pool(x.reshape(-1), batch.astype(jnp.int32))
    return _mlp(u,
                pmax.reshape(NW, B, D), psum.reshape(NW, B, D),
                pcnt.reshape(NW, B),
                W1.T, b1, g1, be1, W2.T, b2, g2, be2)


# speedup vs baseline: 20.4872x; 1.5990x over previous
import functools

import jax
import jax.numpy as jnp
from jax import lax
from jax.experimental import pallas as pl
from jax.experimental.pallas import tpu as pltpu
from jax.experimental.pallas import tpu_sc as plsc

N, D, B, U, HS = 320000, 128, 16, 64, 256
EPS = 1e-5
L = 16
KD = D // L
NC, NS = 2, 16
NW = NC * NS
ROWS_PER_TILE = N // NW
CHUNK_ROWS = 400
NCHUNK = ROWS_PER_TILE // CHUNK_ROWS
GROUPS = CHUNK_ROWS // L

_NEG_INF = float("-inf")


def _pool_body(x_hbm, ids_hbm, omax_hbm, osum_hbm, ocnt_hbm,
               ids_v, xbuf0, xbuf1, accm, accs, accc, sem0, sem1):
    wid = lax.axis_index("c") * NS + lax.axis_index("s")
    base = wid * ROWS_PER_TILE

    def _chunk_src(c):
        return x_hbm.at[pl.ds((base + c * CHUNK_ROWS) * D, CHUNK_ROWS * D)]

    pltpu.async_copy(_chunk_src(0), xbuf0, sem0)

    neg = jnp.full((L,), _NEG_INF, jnp.float32)
    zero = jnp.zeros((L,), jnp.float32)
    for b in range(B):
        for k in range(KD):
            accm[pl.ds((b * KD + k) * L, L)] = neg
            accs[pl.ds((b * KD + k) * L, L)] = zero
    accc[...] = zero

    pltpu.sync_copy(ids_hbm.at[pl.ds(base, ROWS_PER_TILE)], ids_v)

    iota = lax.iota(jnp.int32, L)

    def _merge_rows(seg, m_regs, s_regs, nrows):
        off = seg * D
        for k in range(KD):
            sl = pl.ds(off + k * L, L)
            accm[sl] = jnp.maximum(accm[sl], m_regs[k])
            accs[sl] = accs[sl] + s_regs[k]
        accc[...] = accc[...] + jnp.where(iota == seg, nrows, 0.0)

    def _group(xbuf, c, g):
        row0 = c * CHUNK_ROWS + g * L
        ids = ids_v[pl.ds(row0, L)]
        s_first = ids[0]
        s_last = ids[L - 1]

        def uniform(_):
            lrow = g * L * D
            m_regs = [xbuf[pl.ds(lrow + k * L, L)] for k in range(KD)]
            s_regs = list(m_regs)
            for r in range(1, L):
                roff = lrow + r * D
                for k in range(KD):
                    v = xbuf[pl.ds(roff + k * L, L)]
                    m_regs[k] = jnp.maximum(m_regs[k], v)
                    s_regs[k] = s_regs[k] + v
            _merge_rows(s_first, m_regs, s_regs, jnp.float32(L))
            return 0

        def mixed(_):
            for r in range(L):
                seg = ids[r]
                roff = (g * L + r) * D
                regs = [xbuf[pl.ds(roff + k * L, L)] for k in range(KD)]
                _merge_rows(seg, regs, regs, jnp.float32(1))
            return 0

        lax.cond(s_first == s_last, uniform, mixed, 0)
        return g

    def _process(xbuf, c):
        def g_body(g, _c):
            _group(xbuf, c, g)
            return _c
        lax.fori_loop(0, GROUPS, g_body, 0)

    def _wait(xbuf, sem):
        pltpu.make_async_copy(_chunk_src(0), xbuf, sem).wait()

    def pair_body(p, carry):
        c0 = p * 2
        pltpu.async_copy(_chunk_src(c0 + 1), xbuf1, sem1)
        _wait(xbuf0, sem0)
        _process(xbuf0, c0)
        pltpu.async_copy(_chunk_src(c0 + 2), xbuf0, sem0)
        _wait(xbuf1, sem1)
        _process(xbuf1, c0 + 1)
        return carry

    lax.fori_loop(0, (NCHUNK - 1) // 2, pair_body, 0)
    _wait(xbuf0, sem0)
    _process(xbuf0, NCHUNK - 1)

    pltpu.sync_copy(accm, omax_hbm.at[pl.ds(wid * B * D, B * D)])
    pltpu.sync_copy(accs, osum_hbm.at[pl.ds(wid * B * D, B * D)])
    pltpu.sync_copy(accc, ocnt_hbm.at[pl.ds(wid * B, B)])


_pool = pl.kernel(
    _pool_body,
    out_type=(
        jax.ShapeDtypeStruct((NW * B * D,), jnp.float32),
        jax.ShapeDtypeStruct((NW * B * D,), jnp.float32),
        jax.ShapeDtypeStruct((NW * B,), jnp.float32),
    ),
    mesh=plsc.VectorSubcoreMesh(
        core_axis_name="c", subcore_axis_name="s",
        num_cores=NC, num_subcores=NS),
    scratch_types=[
        pltpu.VMEM((ROWS_PER_TILE,), jnp.int32),
        pltpu.VMEM((CHUNK_ROWS * D,), jnp.float32),
        pltpu.VMEM((CHUNK_ROWS * D,), jnp.float32),
        pltpu.VMEM((B * D,), jnp.float32),
        pltpu.VMEM((B * D,), jnp.float32),
        pltpu.VMEM((B,), jnp.float32),
        pltpu.SemaphoreType.DMA,
        pltpu.SemaphoreType.DMA,
    ],
)


def _mlp_body(u_ref, pmax_ref, psum_ref, pcnt_ref,
              W1t_ref, b1_ref, g1_ref, be1_ref,
              W2t_ref, b2_ref, g2_ref, be2_ref, o_ref):
    smax = jnp.max(pmax_ref[...], axis=0)
    ssum = jnp.sum(psum_ref[...], axis=0)
    cnt = jnp.sum(pcnt_ref[...], axis=0)
    smean = ssum / jnp.maximum(cnt, 1.0)[:, None]
    out = jnp.concatenate([u_ref[...], smax, smean, ssum], axis=1)
    h = lax.dot_general(out, W1t_ref[...], (((1,), (0,)), ((), ())),
                        preferred_element_type=jnp.float32) + b1_ref[...]
    h = (h / jnp.sqrt(1.0 + EPS)) * g1_ref[...] + be1_ref[...]
    h = jnp.maximum(h, 0.0)
    h = lax.dot_general(h, W2t_ref[...], (((1,), (0,)), ((), ())),
                        preferred_element_type=jnp.float32) + b2_ref[...]
    o_ref[...] = (h / jnp.sqrt(1.0 + EPS)) * g2_ref[...] + be2_ref[...]


_mlp = pl.pallas_call(
    _mlp_body,
    out_shape=jax.ShapeDtypeStruct((B, HS), jnp.float32),
)


@jax.jit
def kernel(x, u, batch, W1, b1, g1, be1, W2, b2, g2, be2):
    pmax, psum, pcnt = _pool(x.reshape(-1), batch.astype(jnp.int32))
    return _mlp(u,
                pmax.reshape(NW, B, D), psum.reshape(NW, B, D),
                pcnt.reshape(NW, B),
                W1.T, b1, g1, be1, W2.T, b2, g2, be2)


# speedup vs baseline: 22.4597x; 1.0963x over previous
import functools

import jax
import jax.numpy as jnp
from jax import lax
from jax.experimental import pallas as pl
from jax.experimental.pallas import tpu as pltpu
from jax.experimental.pallas import tpu_sc as plsc

N, D, B, U, HS = 320000, 128, 16, 64, 256
EPS = 1e-5
L = 16
KD = D // L
NC, NS = 2, 16
NW = NC * NS
ROWS_PER_TILE = N // NW
CHUNK_ROWS = 400
NCHUNK = ROWS_PER_TILE // CHUNK_ROWS
GROUPS = CHUNK_ROWS // L

_NEG_INF = float("-inf")


def _pool_body(x_hbm, ids_hbm, omax_hbm, osum_hbm, ocnt_hbm,
               ids_v, xbuf0, xbuf1, accm, accs, accc, sem0, sem1):
    wid = lax.axis_index("c") * NS + lax.axis_index("s")
    base = wid * ROWS_PER_TILE

    def _chunk_src(c):
        return x_hbm.at[pl.ds((base + c * CHUNK_ROWS) * D, CHUNK_ROWS * D)]

    pltpu.async_copy(_chunk_src(0), xbuf0, sem0)

    neg = jnp.full((L,), _NEG_INF, jnp.float32)
    zero = jnp.zeros((L,), jnp.float32)
    for b in range(B):
        for k in range(KD):
            accm[pl.ds((b * KD + k) * L, L)] = neg
            accs[pl.ds((b * KD + k) * L, L)] = zero
    accc[...] = zero

    pltpu.sync_copy(ids_hbm.at[pl.ds(base, ROWS_PER_TILE)], ids_v)

    iota = lax.iota(jnp.int32, L)

    def _merge_rows(seg, m_regs, s_regs, nrows):
        off = seg * D
        for k in range(KD):
            sl = pl.ds(off + k * L, L)
            accm[sl] = jnp.maximum(accm[sl], m_regs[k])
            accs[sl] = accs[sl] + s_regs[k]
        accc[...] = accc[...] + jnp.where(iota == seg, nrows, 0.0)

    def _group(xbuf, c, g):
        row0 = c * CHUNK_ROWS + g * L
        ids = ids_v[pl.ds(row0, L)]
        s_first = ids[0]
        s_last = ids[L - 1]

        def uniform(_):
            lrow = g * L * D
            m_regs = [xbuf[pl.ds(lrow + k * L, L)] for k in range(KD)]
            s_regs = list(m_regs)
            for r in range(1, L):
                roff = lrow + r * D
                for k in range(KD):
                    v = xbuf[pl.ds(roff + k * L, L)]
                    m_regs[k] = jnp.maximum(m_regs[k], v)
                    s_regs[k] = s_regs[k] + v
            _merge_rows(s_first, m_regs, s_regs, jnp.float32(L))
            return 0

        def mixed(_):
            for r in range(L):
                seg = ids[r]
                roff = (g * L + r) * D
                regs = [xbuf[pl.ds(roff + k * L, L)] for k in range(KD)]
                _merge_rows(seg, regs, regs, jnp.float32(1))
            return 0

        lax.cond(s_first == s_last, uniform, mixed, 0)
        return g

    def _process(xbuf, c):
        cbase = c * CHUNK_ROWS
        first = ids_v[pl.ds(cbase, L)][0]
        last = ids_v[pl.ds(cbase + CHUNK_ROWS - L, L)][L - 1]

        def uniform_chunk(_):
            UN = 8
            m_regs = [xbuf[pl.ds(k * L, L)] for k in range(KD)]
            s_regs = list(m_regs)
            for r in range(1, UN):
                for k in range(KD):
                    v = xbuf[pl.ds(r * D + k * L, L)]
                    m_regs[k] = jnp.maximum(m_regs[k], v)
                    s_regs[k] = s_regs[k] + v

            def blk(bi, carry):
                m = list(carry[:KD])
                s = list(carry[KD:])
                boff = bi * (UN * D)
                for r in range(UN):
                    for k in range(KD):
                        v = xbuf[pl.ds(boff + r * D + k * L, L)]
                        m[k] = jnp.maximum(m[k], v)
                        s[k] = s[k] + v
                return tuple(m + s)

            carry = lax.fori_loop(1, CHUNK_ROWS // UN, blk,
                                  tuple(m_regs + s_regs))
            _merge_rows(first, carry[:KD], carry[KD:],
                        jnp.float32(CHUNK_ROWS))
            return 0

        def mixed_chunk(_):
            def g_body(g, _c):
                _group(xbuf, c, g)
                return _c
            lax.fori_loop(0, GROUPS, g_body, 0)
            return 0

        lax.cond(first == last, uniform_chunk, mixed_chunk, 0)

    def _wait(xbuf, sem):
        pltpu.make_async_copy(_chunk_src(0), xbuf, sem).wait()

    def pair_body(p, carry):
        c0 = p * 2
        pltpu.async_copy(_chunk_src(c0 + 1), xbuf1, sem1)
        _wait(xbuf0, sem0)
        _process(xbuf0, c0)
        pltpu.async_copy(_chunk_src(c0 + 2), xbuf0, sem0)
        _wait(xbuf1, sem1)
        _process(xbuf1, c0 + 1)
        return carry

    lax.fori_loop(0, (NCHUNK - 1) // 2, pair_body, 0)
    _wait(xbuf0, sem0)
    _process(xbuf0, NCHUNK - 1)

    pltpu.sync_copy(accm, omax_hbm.at[pl.ds(wid * B * D, B * D)])
    pltpu.sync_copy(accs, osum_hbm.at[pl.ds(wid * B * D, B * D)])
    pltpu.sync_copy(accc, ocnt_hbm.at[pl.ds(wid * B, B)])


_pool = pl.kernel(
    _pool_body,
    out_type=(
        jax.ShapeDtypeStruct((NW * B * D,), jnp.float32),
        jax.ShapeDtypeStruct((NW * B * D,), jnp.float32),
        jax.ShapeDtypeStruct((NW * B,), jnp.float32),
    ),
    mesh=plsc.VectorSubcoreMesh(
        core_axis_name="c", subcore_axis_name="s",
        num_cores=NC, num_subcores=NS),
    scratch_types=[
        pltpu.VMEM((ROWS_PER_TILE,), jnp.int32),
        pltpu.VMEM((CHUNK_ROWS * D,), jnp.float32),
        pltpu.VMEM((CHUNK_ROWS * D,), jnp.float32),
        pltpu.VMEM((B * D,), jnp.float32),
        pltpu.VMEM((B * D,), jnp.float32),
        pltpu.VMEM((B,), jnp.float32),
        pltpu.SemaphoreType.DMA,
        pltpu.SemaphoreType.DMA,
    ],
)


def _mlp_body(u_ref, pmax_ref, psum_ref, pcnt_ref,
              W1t_ref, b1_ref, g1_ref, be1_ref,
              W2t_ref, b2_ref, g2_ref, be2_ref, o_ref):
    smax = jnp.max(pmax_ref[...], axis=0)
    ssum = jnp.sum(psum_ref[...], axis=0)
    cnt = jnp.sum(pcnt_ref[...], axis=0)
    smean = ssum / jnp.maximum(cnt, 1.0)[:, None]
    out = jnp.concatenate([u_ref[...], smax, smean, ssum], axis=1)
    h = lax.dot_general(out, W1t_ref[...], (((1,), (0,)), ((), ())),
                        preferred_element_type=jnp.float32) + b1_ref[...]
    h = (h / jnp.sqrt(1.0 + EPS)) * g1_ref[...] + be1_ref[...]
    h = jnp.maximum(h, 0.0)
    h = lax.dot_general(h, W2t_ref[...], (((1,), (0,)), ((), ())),
                        preferred_element_type=jnp.float32) + b2_ref[...]
    o_ref[...] = (h / jnp.sqrt(1.0 + EPS)) * g2_ref[...] + be2_ref[...]


_mlp = pl.pallas_call(
    _mlp_body,
    out_shape=jax.ShapeDtypeStruct((B, HS), jnp.float32),
)


@jax.jit
def kernel(x, u, batch, W1, b1, g1, be1, W2, b2, g2, be2):
    pmax, psum, pcnt = _pool(x.reshape(-1), batch.astype(jnp.int32))
    return _mlp(u,
                pmax.reshape(NW, B, D), psum.reshape(NW, B, D),
                pcnt.reshape(NW, B),
                W1.T, b1, g1, be1, W2.T, b2, g2, be2)
